# Initial kernel scaffold; baseline (speedup 1.0000x reference)
import math

import jax
import jax.numpy as jnp
from jax.experimental import pallas as pl
from jax.experimental.pallas import tpu as pltpu

_H = 16
_TOTAL_B = 64
_QL = 2048
_KL = 2048
_LINE = 4096
_MW = 3968
_QBLK = 128


def _bucket_row(delta):
    i = jax.lax.broadcasted_iota(jnp.int32, (1, _LINE), 1)
    rel = i - jnp.int32(_KL - 1) + delta
    sign = (rel > 0).astype(jnp.int32)
    relab = jnp.abs(rel)
    max_exact = _TOTAL_B // 4
    is_small = relab < max_exact
    large = max_exact + (
        jnp.log(relab.astype(jnp.float32) / max_exact + 1e-06)
        / math.log(128 / max_exact)
        * (_TOTAL_B // 2 - max_exact)
    ).astype(jnp.int32)
    large = jnp.minimum(large, _TOTAL_B // 2 - 1)
    buckets = jnp.where(is_small, relab, large)
    buckets = buckets + sign * (_TOTAL_B // 2)
    return jnp.clip(buckets, 0, _TOTAL_B - 1)


def _body(delta_ref, emb_ref, out_ref, lines_ref, m_ref):
    h = pl.program_id(0)
    a = pl.program_id(1)

    @pl.when(jnp.logical_and(h == 0, a == 0))
    def _():
        buckets = _bucket_row(delta_ref[0])
        acc = jnp.zeros((_H, _LINE), jnp.float32)
        for b in range(_TOTAL_B):
            col = emb_ref[b, :].reshape(_H, 1)
            acc = jnp.where(buckets == b, col, acc)
        lines_ref[...] = acc

    @pl.when(a == 0)
    def _():
        def build(s, carry):
            m_ref[s, :] = lines_ref[h, pl.ds(127 - s, _MW)]
            return carry
        jax.lax.fori_loop(0, _QBLK, build, 0)

    u0 = pl.multiple_of((_QL - _QBLK) - _QBLK * a, _QBLK)
    out_ref[0] = m_ref[:, pl.ds(u0, _KL)]


def kernel(q_len, k_len, emb):
    delta = (jnp.asarray(k_len, jnp.int32) - jnp.asarray(q_len, jnp.int32)).reshape(1)
    out = pl.pallas_call(
        _body,
        grid=(_H, _QL // _QBLK),
        in_specs=[
            pl.BlockSpec(memory_space=pltpu.SMEM),
            pl.BlockSpec((_TOTAL_B, _H), lambda h, a: (0, 0)),
        ],
        out_specs=pl.BlockSpec((1, _QBLK, _KL), lambda h, a: (h, a, 0)),
        out_shape=jax.ShapeDtypeStruct((_H, _QL, _KL), jnp.float32),
        scratch_shapes=[
            pltpu.VMEM((_H, _LINE), jnp.float32),
            pltpu.VMEM((_QBLK, _MW), jnp.float32),
        ],
    )(delta, emb)
    return out[None]


# speedup vs baseline: 41.7538x; 41.7538x over previous
import math

import jax
import jax.numpy as jnp
from jax.experimental import pallas as pl
from jax.experimental.pallas import tpu as pltpu

_H = 16
_TOTAL_B = 64
_QL = 2048
_KL = 2048
_LINE = 4096
_MW = 3968
_QBLK = 128


def _bucket_row(delta):
    i = jax.lax.broadcasted_iota(jnp.int32, (1, _LINE), 1)
    rel = i - jnp.int32(_KL - 1) + delta
    sign = (rel > 0).astype(jnp.int32)
    relab = jnp.abs(rel)
    max_exact = _TOTAL_B // 4
    is_small = relab < max_exact
    large = max_exact + (
        jnp.log(relab.astype(jnp.float32) / max_exact + 1e-06)
        / math.log(128 / max_exact)
        * (_TOTAL_B // 2 - max_exact)
    ).astype(jnp.int32)
    large = jnp.minimum(large, _TOTAL_B // 2 - 1)
    buckets = jnp.where(is_small, relab, large)
    buckets = buckets + sign * (_TOTAL_B // 2)
    return jnp.clip(buckets, 0, _TOTAL_B - 1)


def _body(delta_ref, emb_ref, out_ref, lines_ref, m_ref):
    h = pl.program_id(0)
    a = pl.program_id(1)

    @pl.when(jnp.logical_and(h == 0, a == 0))
    def _():
        buckets = _bucket_row(delta_ref[0])
        acc = jnp.zeros((_H, _LINE), jnp.float32)
        for b in range(_TOTAL_B):
            col = emb_ref[b, :].reshape(_H, 1)
            acc = jnp.where(buckets == b, col, acc)
        lines_ref[...] = acc

    @pl.when(a == 0)
    def _():
        row = lines_ref[h, :].reshape(1, _LINE)

        def build(s, carry):
            rolled = pltpu.roll(row, s + (_LINE - 127), 1)
            m_ref[s, :] = rolled[0, :_MW]
            return carry
        jax.lax.fori_loop(0, _QBLK, build, 0)

    u0 = pl.multiple_of((_QL - _QBLK) - _QBLK * a, _QBLK)
    out_ref[0] = m_ref[:, pl.ds(u0, _KL)]


def kernel(q_len, k_len, emb):
    delta = (jnp.asarray(k_len, jnp.int32) - jnp.asarray(q_len, jnp.int32)).reshape(1)
    out = pl.pallas_call(
        _body,
        grid=(_H, _QL // _QBLK),
        in_specs=[
            pl.BlockSpec(memory_space=pltpu.SMEM),
            pl.BlockSpec((_TOTAL_B, _H), lambda h, a: (0, 0)),
        ],
        out_specs=pl.BlockSpec((1, _QBLK, _KL), lambda h, a: (h, a, 0)),
        out_shape=jax.ShapeDtypeStruct((_H, _QL, _KL), jnp.float32),
        scratch_shapes=[
            pltpu.VMEM((_H, _LINE), jnp.float32),
            pltpu.VMEM((_QBLK, _MW), jnp.float32),
        ],
    )(delta, emb)
    return out[None]


# speedup vs baseline: 105.0604x; 2.5162x over previous
import math

import jax
import jax.numpy as jnp
from jax.experimental import pallas as pl
from jax.experimental.pallas import tpu as pltpu

_H = 16
_TOTAL_B = 64
_QL = 2048
_KL = 2048
_LINE = 4096
_MW = 3968
_QBLK = 128


def _bucket_row(delta):
    i = jax.lax.broadcasted_iota(jnp.int32, (1, _LINE), 1)
    rel = i - jnp.int32(_KL - 1) + delta
    sign = (rel > 0).astype(jnp.int32)
    relab = jnp.abs(rel)
    max_exact = _TOTAL_B // 4
    is_small = relab < max_exact
    large = max_exact + (
        jnp.log(relab.astype(jnp.float32) / max_exact + 1e-06)
        / math.log(128 / max_exact)
        * (_TOTAL_B // 2 - max_exact)
    ).astype(jnp.int32)
    large = jnp.minimum(large, _TOTAL_B // 2 - 1)
    buckets = jnp.where(is_small, relab, large)
    buckets = buckets + sign * (_TOTAL_B // 2)
    return jnp.clip(buckets, 0, _TOTAL_B - 1)


def _body(delta_ref, emb_ref, out_ref, lines_ref, m_ref):
    h = pl.program_id(0)
    a = pl.program_id(1)

    @pl.when(jnp.logical_and(h == 0, a == 0))
    def _():
        buckets = _bucket_row(delta_ref[0])
        acc = jnp.zeros((_H, _LINE), jnp.float32)
        for b in range(_TOTAL_B):
            col = emb_ref[b, :].reshape(_H, 1)
            acc = jnp.where(buckets == b, col, acc)
        lines_ref[...] = acc

    @pl.when(a == 0)
    def _():
        m_ref[_QBLK - 1, :] = lines_ref[h, :]
        for j in range(7):
            w = 1 << j
            src = m_ref[_QBLK - w:_QBLK, :]
            m_ref[_QBLK - 2 * w:_QBLK - w, :] = pltpu.roll(src, _LINE - w, 1)

    u0 = pl.multiple_of((_QL - _QBLK) - _QBLK * a, _QBLK)
    out_ref[0] = m_ref[:, pl.ds(u0, _KL)]


def kernel(q_len, k_len, emb):
    delta = (jnp.asarray(k_len, jnp.int32) - jnp.asarray(q_len, jnp.int32)).reshape(1)
    out = pl.pallas_call(
        _body,
        grid=(_H, _QL // _QBLK),
        in_specs=[
            pl.BlockSpec(memory_space=pltpu.SMEM),
            pl.BlockSpec((_TOTAL_B, _H), lambda h, a: (0, 0)),
        ],
        out_specs=pl.BlockSpec((1, _QBLK, _KL), lambda h, a: (h, a, 0)),
        out_shape=jax.ShapeDtypeStruct((_H, _QL, _KL), jnp.float32),
        scratch_shapes=[
            pltpu.VMEM((_H, _LINE), jnp.float32),
            pltpu.VMEM((_QBLK, _LINE), jnp.float32),
        ],
    )(delta, emb)
    return out[None]


# speedup vs baseline: 176.6311x; 1.6812x over previous
import math

import jax
import jax.numpy as jnp
from jax.experimental import pallas as pl
from jax.experimental.pallas import tpu as pltpu

_H = 16
_TOTAL_B = 64
_QL = 2048
_KL = 2048
_LINE = 4096
_MW = 3968
_QBLK = 128


def _bucket_row(delta):
    i = jax.lax.broadcasted_iota(jnp.int32, (1, _LINE), 1)
    rel = i - jnp.int32(_KL - 1) + delta
    sign = (rel > 0).astype(jnp.int32)
    relab = jnp.abs(rel)
    max_exact = _TOTAL_B // 4
    is_small = relab < max_exact
    large = max_exact + (
        jnp.log(relab.astype(jnp.float32) / max_exact + 1e-06)
        / math.log(128 / max_exact)
        * (_TOTAL_B // 2 - max_exact)
    ).astype(jnp.int32)
    large = jnp.minimum(large, _TOTAL_B // 2 - 1)
    buckets = jnp.where(is_small, relab, large)
    buckets = buckets + sign * (_TOTAL_B // 2)
    return jnp.clip(buckets, 0, _TOTAL_B - 1)


_NSEM = 4


def _body(delta_ref, emb_ref, out_ref, lines_ref, m_ref, sems):
    h = pl.program_id(0)
    a = pl.program_id(1)
    i = h * (_QL // _QBLK) + a
    n = _H * (_QL // _QBLK)

    @pl.when(i == 0)
    def _():
        buckets = _bucket_row(delta_ref[0])
        acc = jnp.zeros((_H, _LINE), jnp.float32)
        for b in range(_TOTAL_B):
            col = emb_ref[b, :].reshape(_H, 1)
            acc = jnp.where(buckets == b, col, acc)
        lines_ref[...] = acc

    mh = m_ref.at[h % 2]

    @pl.when(a == 0)
    def _():
        mh[_QBLK - 1, :] = lines_ref[h, :]
        for j in range(7):
            w = 1 << j
            src = mh[_QBLK - w:_QBLK, :]
            mh[_QBLK - 2 * w:_QBLK - w, :] = pltpu.roll(src, _LINE - w, 1)

    u0 = pl.multiple_of((_QL - _QBLK) - _QBLK * a, _QBLK)
    src = mh.at[:, pl.ds(u0, _KL)]
    dst = out_ref.at[h, pl.ds(a * _QBLK, _QBLK), :]

    @pl.when(i >= _NSEM)
    def _():
        pltpu.make_async_copy(src, dst, sems.at[i % _NSEM]).wait()

    pltpu.make_async_copy(src, dst, sems.at[i % _NSEM]).start()

    @pl.when(i == n - 1)
    def _():
        for t in range(_NSEM):
            pltpu.make_async_copy(src, dst, sems.at[t]).wait()


def kernel(q_len, k_len, emb):
    delta = (jnp.asarray(k_len, jnp.int32) - jnp.asarray(q_len, jnp.int32)).reshape(1)
    out = pl.pallas_call(
        _body,
        grid=(_H, _QL // _QBLK),
        in_specs=[
            pl.BlockSpec(memory_space=pltpu.SMEM),
            pl.BlockSpec((_TOTAL_B, _H), lambda h, a: (0, 0)),
        ],
        out_specs=pl.BlockSpec(memory_space=pl.ANY),
        out_shape=jax.ShapeDtypeStruct((_H, _QL, _KL), jnp.float32),
        scratch_shapes=[
            pltpu.VMEM((_H, _LINE), jnp.float32),
            pltpu.VMEM((2, _QBLK, _LINE), jnp.float32),
            pltpu.SemaphoreType.DMA((_NSEM,)),
        ],
    )(delta, emb)
    return out[None]
